# baseline (speedup 1.0000x reference)
import functools
import math

import jax
import jax.numpy as jnp
from jax import lax
from jax.experimental import pallas as pl
from jax.experimental.pallas import tpu as pltpu
from jax.experimental.pallas import tpu_sc as plsc

HID = 128
LAT = 256
N_RBF = 16
CUTOFF = 6.0
MAXA = 100

C_PAD = 512
APC = 20
A_PAD = C_PAD * APC
C_BLK = 128
A_BLK = C_BLK * APC

E_ROWS = 2560
E_PAD = E_ROWS * 128
E_BLK = 4096
PAD_ATOM = 10008
A_CART = 10048
NQ = A_PAD // 128

NW = 32
RPW = E_ROWS // NW
SPAIR = E_ROWS // 16
R_C0 = 40
R_C1 = SPAIR - R_C0
TPS = 16
A_SLC = A_PAD // TPS

_F32 = jnp.float32


def _atom_kernel(types_ref, frac_ref, z_ref, len_ref, ang_ref, emb_ref,
                 wz_ref, bz_ref, wsrc_ref, wdst_ref,
                 h_ref, a_ref, b_ref, cart_ref):
    f32 = _F32
    lanes_a = jax.lax.broadcasted_iota(jnp.int32, (A_BLK, 128), 1)
    t = types_ref[:]
    oh_t = (lanes_a == t).astype(f32)
    emb = jnp.dot(oh_t, emb_ref[:], preferred_element_type=f32)

    zw = jnp.dot(z_ref[:], wz_ref[:], preferred_element_type=f32)
    rows = jax.lax.broadcasted_iota(jnp.int32, (A_BLK, 128), 0)
    oh_c = (lanes_a == rows // APC).astype(f32)
    zw_at = jnp.dot(oh_c, zw, preferred_element_type=f32)

    h = emb + zw_at + bz_ref[:]
    h_ref[:] = h
    a_ref[:] = jnp.dot(h, wsrc_ref[:], preferred_element_type=f32)
    b_ref[:] = jnp.dot(h, wdst_ref[:], preferred_element_type=f32)

    ang = ang_ref[:] * (math.pi / 180.0)
    cos_a = jnp.cos(ang[:, 0:1])
    cos_b = jnp.cos(ang[:, 1:2])
    cos_g = jnp.cos(ang[:, 2:3])
    sin_g = jnp.clip(jnp.sin(ang[:, 2:3]), 1e-6, None)
    a_len = len_ref[:, 0:1]
    b_len = len_ref[:, 1:2]
    c_len = len_ref[:, 2:3]
    zeros = jnp.zeros_like(a_len)
    cx = c_len * cos_b
    cy = c_len * (cos_a - cos_b * cos_g) / sin_g
    cz = jnp.sqrt(jnp.clip(c_len * c_len - cx * cx - cy * cy, 1e-6, None))
    lrow = jnp.concatenate(
        [a_len, zeros, zeros,
         b_len * cos_g, b_len * sin_g, zeros,
         cx, cy, cz,
         zeros, zeros, zeros, zeros, zeros, zeros, zeros], axis=1)
    l_at = jnp.dot(oh_c, lrow, preferred_element_type=f32)

    f = frac_ref[:]
    f0, f1, f2 = f[:, 0:1], f[:, 1:2], f[:, 2:3]
    cart_x = f0 * l_at[:, 0:1] + f1 * l_at[:, 3:4] + f2 * l_at[:, 6:7]
    cart_y = f0 * l_at[:, 1:2] + f1 * l_at[:, 4:5] + f2 * l_at[:, 7:8]
    cart_z = f0 * l_at[:, 2:3] + f1 * l_at[:, 5:6] + f2 * l_at[:, 8:9]
    zpad = jnp.zeros((A_BLK, 1), f32)
    cart_ref[:] = jnp.concatenate([cart_x, cart_y, cart_z, zpad], axis=1)


def _sc_gather(a_h, b_h, cart_h, src_h, dst_h, lanec_h,
               esum_h, vec_h,
               sidx, didx, consts, buf_a0, buf_b0, buf_a1, buf_b1,
               cartv, vbuf0, vbuf1, sem0, sem1, semv, semi):
    f32 = _F32
    i32 = jnp.int32
    cid = lax.axis_index("c")
    sid = lax.axis_index("s")
    base = sid * SPAIR + cid * R_C0
    n_oct = (R_C0 // 8) + cid * ((R_C1 - R_C0) // 8)
    pltpu.sync_copy(cart_h, cartv)
    pltpu.sync_copy(lanec_h, consts)

    def make_vec(ro, vbuf):
        ld4 = consts[0]
        lm4 = consts[1]
        for k in range(32):
            ei = ld4 + (k * 4 + ro * 128)
            sa = plsc.load_gather(sidx, [ei])
            da = plsc.load_gather(didx, [ei])
            cs = plsc.load_gather(cartv, [sa * 4 + lm4])
            cd = plsc.load_gather(cartv, [da * 4 + lm4])
            vbuf[pl.ds(k * 16, 16)] = cs - cd

    def add_into(dst_buf, src_buf):
        def row(j2, _):
            for k in range(8):
                dst_buf[j2, pl.ds(k * 16, 16)] = (
                    dst_buf[j2, pl.ds(k * 16, 16)]
                    + src_buf[j2, pl.ds(k * 16, 16)])
            return 0
        lax.fori_loop(0, 128, row, 0)

    def octet(oo, _):
        g = base + 8 * oo
        ci0 = pltpu.async_copy(src_h.at[pl.ds(g * 128, 1024)], sidx, semi)
        ci1 = pltpu.async_copy(dst_h.at[pl.ds(g * 128, 1024)], didx, semi)
        ci0.wait(); ci1.wait()
        for pp in range(4):
            ro0 = 2 * pp
            ro1 = ro0 + 1
            r0 = (g + ro0) * 128
            r1 = (g + ro1) * 128
            g0a = pltpu.async_copy(a_h.at[sidx.at[pl.ds(ro0 * 128, 128)]],
                                   buf_a0, sem0)
            g0b = pltpu.async_copy(b_h.at[didx.at[pl.ds(ro0 * 128, 128)]],
                                   buf_b0, sem0)
            g1a = pltpu.async_copy(a_h.at[sidx.at[pl.ds(ro1 * 128, 128)]],
                                   buf_a1, sem1)
            g1b = pltpu.async_copy(b_h.at[didx.at[pl.ds(ro1 * 128, 128)]],
                                   buf_b1, sem1)
            make_vec(ro0, vbuf0)
            wv0 = pltpu.async_copy(vbuf0, vec_h.at[pl.ds(r0 * 4, 512)], semv)
            make_vec(ro1, vbuf1)
            wv1 = pltpu.async_copy(vbuf1, vec_h.at[pl.ds(r1 * 4, 512)], semv)
            g0a.wait(); g0b.wait()
            add_into(buf_a0, buf_b0)
            w0 = pltpu.async_copy(buf_a0, esum_h.at[pl.ds(r0, 128)], sem0)
            g1a.wait(); g1b.wait()
            add_into(buf_a1, buf_b1)
            w1 = pltpu.async_copy(buf_a1, esum_h.at[pl.ds(r1, 128)], sem1)
            w0.wait(); w1.wait()
            wv0.wait(); wv1.wait()
        return 0

    lax.fori_loop(0, n_oct, octet, 0)


def _edge_kernel(es_ref, vec_ref, dstc_ref, wrbf_ref, bmsg_ref,
                 wf_ref, m_ref, u_ref):
    f32 = _F32
    i = pl.program_id(0)
    vec = vec_ref[:]
    d2 = jnp.sum(vec * vec, axis=1, keepdims=True)
    dist = jnp.sqrt(d2) + 1e-8
    centers = jax.lax.broadcasted_iota(jnp.int32, (E_BLK, N_RBF), 1).astype(f32) * (
        CUTOFF / (N_RBF - 1))
    rbf = jnp.exp(-((dist - centers) ** 2) / 0.5)
    rbf_w = jnp.dot(rbf, wrbf_ref[:], preferred_element_type=f32)
    mpre = es_ref[:] + rbf_w + bmsg_ref[:]
    m = mpre / (1.0 + jnp.exp(-mpre))
    m_ref[:] = m
    fcoef = jnp.sum(m * wf_ref[:], axis=1, keepdims=True)
    fv = fcoef * (vec / dist)

    dstc = dstc_ref[:]
    r_e = dstc % 128
    q_e = dstc // 128
    oh_r = (jax.lax.broadcasted_iota(jnp.int32, (E_BLK, 128), 1) == r_e
            ).astype(f32)
    oh_q = (jax.lax.broadcasted_iota(jnp.int32, (E_BLK, NQ), 1) == q_e
            ).astype(f32)
    dims = (((0,), (0,)), ((), ()))

    @pl.when(i == 0)
    def _():
        u_ref[:] = jnp.zeros_like(u_ref)

    for c in range(3):
        uc = jax.lax.dot_general(oh_r, oh_q * fv[:, c:c + 1], dims,
                                 preferred_element_type=f32)
        u_ref[c] += uc


def _sc_scatter(m_h, dst_h, z128_h, oagg_h,
                didx, mbuf0, mbuf1, agg_sp, sem0, sem1):
    cid = lax.axis_index("c")
    sid = lax.axis_index("s")
    wid = sid * 2 + cid
    base = wid * RPW
    row0 = sid * A_SLC
    pltpu.sync_copy(z128_h.at[pl.ds(row0, A_SLC)], agg_sp.at[pl.ds(row0, A_SLC)])
    pltpu.sync_copy(dst_h.at[pl.ds(base, RPW)], didx)
    plsc.subcore_barrier()

    def pair(jj, _):
        j0 = 2 * jj
        j1 = j0 + 1
        c0 = pltpu.async_copy(m_h.at[pl.ds((base + j0) * 128, 128)], mbuf0, sem0)
        c1 = pltpu.async_copy(m_h.at[pl.ds((base + j1) * 128, 128)], mbuf1, sem1)
        c0.wait()
        pltpu.sync_copy(mbuf0, agg_sp.at[didx.at[j0]], add=True)
        c1.wait()
        pltpu.sync_copy(mbuf1, agg_sp.at[didx.at[j1]], add=True)
        return 0

    lax.fori_loop(0, RPW // 2, pair, 0)
    plsc.subcore_barrier()
    pltpu.sync_copy(agg_sp.at[pl.ds(row0, A_SLC)],
                    oagg_h.at[cid, pl.ds(row0, A_SLC)])


def _out_kernel(h_ref, agg0_ref, agg1_ref,
                wu1_ref, wu2_ref, bu_ref, watom_ref, ba_ref,
                out_ref):
    f32 = _F32
    agg = agg0_ref[0] + agg1_ref[0]
    pre = (jnp.dot(h_ref[:], wu1_ref[:], preferred_element_type=f32)
           + jnp.dot(agg, wu2_ref[:], preferred_element_type=f32)
           + bu_ref[:])
    h2 = pre / (1.0 + jnp.exp(-pre))
    out_ref[:] = jnp.dot(h2, watom_ref[:], preferred_element_type=f32) + ba_ref[:]


def kernel(z, pred_frac_coords, pred_atom_types, num_atoms, lengths, angles,
           edge_index, atom_emb, W_z, b_z, W_msg, b_msg, W_upd, b_upd, W_f,
           W_atom, b_atom):
    f32 = _F32
    n_cryst = z.shape[0]
    n_atoms = pred_frac_coords.shape[0]
    n_edges = edge_index.shape[1]

    types2d = jnp.clip(pred_atom_types, 0, MAXA).astype(jnp.int32).reshape(-1, 1)
    types2d = jnp.pad(types2d, ((0, A_PAD - n_atoms), (0, 0)))
    frac4 = jnp.pad(pred_frac_coords.astype(f32), ((0, A_PAD - n_atoms), (0, 1)))
    z_p = jnp.pad(z.astype(f32), ((0, C_PAD - n_cryst), (0, 0)))
    len4 = jnp.pad(lengths.astype(f32), ((0, C_PAD - n_cryst), (0, 1)))
    ang4 = jnp.pad(angles.astype(f32), ((0, C_PAD - n_cryst), (0, 1)))
    emb_p = jnp.pad(atom_emb.astype(f32), ((0, 128 - (MAXA + 1)), (0, 0)))
    w_src = W_msg[:HID]
    w_dst = W_msg[HID:2 * HID]
    w_rbf = W_msg[2 * HID:]
    bz_r = b_z.reshape(1, HID)
    bmsg_r = b_msg.reshape(1, HID)
    wf_r = W_f.reshape(1, HID)
    wu1 = W_upd[:HID]
    wu2 = W_upd[HID:]
    bu_r = b_upd.reshape(1, HID)
    watom_p = jnp.pad(W_atom.astype(f32), ((0, 0), (0, 128 - MAXA)))
    ba_p = jnp.pad(b_atom.astype(f32), (0, 128 - MAXA)).reshape(1, 128)

    src_p = jnp.pad(edge_index[0].astype(jnp.int32), (0, E_PAD - n_edges),
                    constant_values=PAD_ATOM)
    dst_p = jnp.pad(edge_index[1].astype(jnp.int32), (0, E_PAD - n_edges),
                    constant_values=PAD_ATOM)
    src2d = src_p.reshape(E_ROWS, 128)
    dst2d = dst_p.reshape(E_ROWS, 128)
    dstc = dst_p.reshape(E_PAD, 1)
    zeros128 = jnp.zeros((A_PAD, HID), f32)

    n_ab = A_PAD // A_BLK

    def full(shp):
        return pl.BlockSpec(shp, lambda i: tuple(0 for _ in shp))

    h, a_tab, b_tab, cart4 = pl.pallas_call(
        _atom_kernel,
        grid=(n_ab,),
        in_specs=[
            pl.BlockSpec((A_BLK, 1), lambda i: (i, 0)),
            pl.BlockSpec((A_BLK, 4), lambda i: (i, 0)),
            pl.BlockSpec((C_BLK, LAT), lambda i: (i, 0)),
            pl.BlockSpec((C_BLK, 4), lambda i: (i, 0)),
            pl.BlockSpec((C_BLK, 4), lambda i: (i, 0)),
            full((128, HID)),
            full((LAT, HID)),
            full((1, HID)),
            full((HID, HID)),
            full((HID, HID)),
        ],
        out_specs=[
            pl.BlockSpec((A_BLK, HID), lambda i: (i, 0)),
            pl.BlockSpec((A_BLK, HID), lambda i: (i, 0)),
            pl.BlockSpec((A_BLK, HID), lambda i: (i, 0)),
            pl.BlockSpec((A_BLK, 4), lambda i: (i, 0)),
        ],
        out_shape=[
            jax.ShapeDtypeStruct((A_PAD, HID), f32),
            jax.ShapeDtypeStruct((A_PAD, HID), f32),
            jax.ShapeDtypeStruct((A_PAD, HID), f32),
            jax.ShapeDtypeStruct((A_PAD, 4), f32),
        ],
    )(types2d, frac4, z_p, len4, ang4, emb_p, W_z, bz_r, w_src, w_dst)

    cart_flat = cart4[:A_CART].reshape(A_CART * 4)

    mesh = plsc.VectorSubcoreMesh(core_axis_name="c", subcore_axis_name="s")
    gather_fn = functools.partial(
        pl.kernel,
        out_type=[
            jax.ShapeDtypeStruct((E_PAD, HID), f32),
            jax.ShapeDtypeStruct((E_PAD * 4,), f32),
        ],
        mesh=mesh,
        compiler_params=pltpu.CompilerParams(needs_layout_passes=False),
        scratch_types=[
            pltpu.VMEM((1024,), jnp.int32),
            pltpu.VMEM((1024,), jnp.int32),
            pltpu.VMEM((2, 16), jnp.int32),
            pltpu.VMEM((128, HID), f32),
            pltpu.VMEM((128, HID), f32),
            pltpu.VMEM((128, HID), f32),
            pltpu.VMEM((128, HID), f32),
            pltpu.VMEM((A_CART * 4,), f32),
            pltpu.VMEM((512,), f32),
            pltpu.VMEM((512,), f32),
            pltpu.SemaphoreType.DMA,
            pltpu.SemaphoreType.DMA,
            pltpu.SemaphoreType.DMA,
            pltpu.SemaphoreType.DMA,
        ],
    )(_sc_gather)
    lane_consts = jnp.array([[0, 0, 0, 0, 1, 1, 1, 1, 2, 2, 2, 2, 3, 3, 3, 3],
                             [0, 1, 2, 3, 0, 1, 2, 3, 0, 1, 2, 3, 0, 1, 2, 3]],
                            jnp.int32)
    e_sum, vec_flat = gather_fn(a_tab, b_tab, cart_flat, src_p, dst_p, lane_consts)
    vec4 = vec_flat.reshape(E_PAD, 4)

    n_eb = E_PAD // E_BLK
    m, u_acc = pl.pallas_call(
        _edge_kernel,
        grid=(n_eb,),
        in_specs=[
            pl.BlockSpec((E_BLK, HID), lambda i: (i, 0)),
            pl.BlockSpec((E_BLK, 4), lambda i: (i, 0)),
            pl.BlockSpec((E_BLK, 1), lambda i: (i, 0)),
            full((N_RBF, HID)),
            full((1, HID)),
            full((1, HID)),
        ],
        out_specs=[
            pl.BlockSpec((E_BLK, HID), lambda i: (i, 0)),
            pl.BlockSpec((3, 128, NQ), lambda i: (0, 0, 0)),
        ],
        out_shape=[
            jax.ShapeDtypeStruct((E_PAD, HID), f32),
            jax.ShapeDtypeStruct((3, 128, NQ), f32),
        ],
    )(e_sum, vec4, dstc, w_rbf, bmsg_r, wf_r)

    scatter_fn = functools.partial(
        pl.kernel,
        out_type=jax.ShapeDtypeStruct((2, A_PAD, HID), f32),
        mesh=mesh,
        scratch_types=[
            pltpu.VMEM((RPW, 128), jnp.int32),
            pltpu.VMEM((128, HID), f32),
            pltpu.VMEM((128, HID), f32),
            pltpu.VMEM_SHARED((A_PAD, HID), f32),
            pltpu.SemaphoreType.DMA,
            pltpu.SemaphoreType.DMA,
        ],
    )(_sc_scatter)
    agg_part = scatter_fn(m, dst2d, zeros128)

    out_p = pl.pallas_call(
        _out_kernel,
        grid=(n_ab,),
        in_specs=[
            pl.BlockSpec((A_BLK, HID), lambda i: (i, 0)),
            pl.BlockSpec((1, A_BLK, HID), lambda i: (0, i, 0)),
            pl.BlockSpec((1, A_BLK, HID), lambda i: (1, i, 0)),
            full((HID, HID)),
            full((HID, HID)),
            full((1, HID)),
            full((HID, 128)),
            full((1, 128)),
        ],
        out_specs=pl.BlockSpec((A_BLK, 128), lambda i: (i, 0)),
        out_shape=jax.ShapeDtypeStruct((A_PAD, 128), f32),
    )(h, agg_part, agg_part, wu1, wu2, bu_r, watom_p, ba_p)

    pred_cart = jnp.transpose(u_acc, (2, 1, 0)).reshape(A_PAD, 3)[:n_atoms]
    pred_atom_types_out = out_p[:n_atoms, :MAXA]
    return (pred_cart, pred_atom_types_out)


# speedup vs baseline: 1.1628x; 1.1628x over previous
import functools
import math

import jax
import jax.numpy as jnp
from jax import lax
from jax.experimental import pallas as pl
from jax.experimental.pallas import tpu as pltpu
from jax.experimental.pallas import tpu_sc as plsc

HID = 128
LAT = 256
N_RBF = 16
CUTOFF = 6.0
MAXA = 100

C_PAD = 512
APC = 20
A_PAD = C_PAD * APC
C_BLK = 128
A_BLK = C_BLK * APC

E_ROWS = 2560
E_PAD = E_ROWS * 128
E_BLK = 4096
PAD_ATOM = 10008
A_CART = 10048
NQ = A_PAD // 128

NW = 32
RPW = E_ROWS // NW
SPAIR = E_ROWS // 16
R_C0 = 120
R_C1 = SPAIR - R_C0
TPS = 16
A_SLC = A_PAD // TPS

_F32 = jnp.float32


def _atom_kernel(types_ref, frac_ref, z_ref, len_ref, ang_ref, emb_ref,
                 wz_ref, bz_ref, wsrc_ref, wdst_ref,
                 h_ref, a_ref, b_ref, cart_ref):
    f32 = _F32
    lanes_a = jax.lax.broadcasted_iota(jnp.int32, (A_BLK, 128), 1)
    t = types_ref[:]
    oh_t = (lanes_a == t).astype(f32)
    emb = jnp.dot(oh_t, emb_ref[:], preferred_element_type=f32)

    zw = jnp.dot(z_ref[:], wz_ref[:], preferred_element_type=f32)
    rows = jax.lax.broadcasted_iota(jnp.int32, (A_BLK, 128), 0)
    oh_c = (lanes_a == rows // APC).astype(f32)
    zw_at = jnp.dot(oh_c, zw, preferred_element_type=f32)

    h = emb + zw_at + bz_ref[:]
    h_ref[:] = h
    a_ref[:] = jnp.dot(h, wsrc_ref[:], preferred_element_type=f32)
    b_ref[:] = jnp.dot(h, wdst_ref[:], preferred_element_type=f32)

    ang = ang_ref[:] * (math.pi / 180.0)
    cos_a = jnp.cos(ang[:, 0:1])
    cos_b = jnp.cos(ang[:, 1:2])
    cos_g = jnp.cos(ang[:, 2:3])
    sin_g = jnp.clip(jnp.sin(ang[:, 2:3]), 1e-6, None)
    a_len = len_ref[:, 0:1]
    b_len = len_ref[:, 1:2]
    c_len = len_ref[:, 2:3]
    zeros = jnp.zeros_like(a_len)
    cx = c_len * cos_b
    cy = c_len * (cos_a - cos_b * cos_g) / sin_g
    cz = jnp.sqrt(jnp.clip(c_len * c_len - cx * cx - cy * cy, 1e-6, None))
    lrow = jnp.concatenate(
        [a_len, zeros, zeros,
         b_len * cos_g, b_len * sin_g, zeros,
         cx, cy, cz,
         zeros, zeros, zeros, zeros, zeros, zeros, zeros], axis=1)
    l_at = jnp.dot(oh_c, lrow, preferred_element_type=f32)

    f = frac_ref[:]
    f0, f1, f2 = f[:, 0:1], f[:, 1:2], f[:, 2:3]
    cart_x = f0 * l_at[:, 0:1] + f1 * l_at[:, 3:4] + f2 * l_at[:, 6:7]
    cart_y = f0 * l_at[:, 1:2] + f1 * l_at[:, 4:5] + f2 * l_at[:, 7:8]
    cart_z = f0 * l_at[:, 2:3] + f1 * l_at[:, 5:6] + f2 * l_at[:, 8:9]
    zpad = jnp.zeros((A_BLK, 1), f32)
    cart_ref[:] = jnp.concatenate([cart_x, cart_y, cart_z, zpad], axis=1)


def _sc_gather(a_h, b_h, cart_h, src_h, dst_h, lanec_h,
               esum_h, vec_h,
               sidx, didx, consts, buf_a0, buf_b0, buf_a1, buf_b1,
               cartv, vbuf0, vbuf1, sem0, sem1, semv, semi):
    f32 = _F32
    i32 = jnp.int32
    cid = lax.axis_index("c")
    sid = lax.axis_index("s")
    base = sid * SPAIR + cid * R_C0
    n_oct = (R_C0 // 8) + cid * ((R_C1 - R_C0) // 8)
    pltpu.sync_copy(cart_h, cartv)
    pltpu.sync_copy(lanec_h, consts)

    def make_vec(ro, vbuf):
        ld4 = consts[0]
        lm4 = consts[1]
        for k in range(32):
            ei = ld4 + (k * 4 + ro * 128)
            sa = plsc.load_gather(sidx, [ei])
            da = plsc.load_gather(didx, [ei])
            cs = plsc.load_gather(cartv, [sa * 4 + lm4])
            cd = plsc.load_gather(cartv, [da * 4 + lm4])
            vbuf[pl.ds(k * 16, 16)] = cs - cd

    def add_into(dst_buf, src_buf):
        def row(j2, _):
            for k in range(8):
                dst_buf[j2, pl.ds(k * 16, 16)] = (
                    dst_buf[j2, pl.ds(k * 16, 16)]
                    + src_buf[j2, pl.ds(k * 16, 16)])
            return 0
        lax.fori_loop(0, 128, row, 0)

    def octet(oo, _):
        g = base + 8 * oo
        ci0 = pltpu.async_copy(src_h.at[pl.ds(g * 128, 1024)], sidx, semi)
        ci1 = pltpu.async_copy(dst_h.at[pl.ds(g * 128, 1024)], didx, semi)
        ci0.wait(); ci1.wait()
        for pp in range(4):
            ro0 = 2 * pp
            ro1 = ro0 + 1
            r0 = (g + ro0) * 128
            r1 = (g + ro1) * 128
            g0a = pltpu.async_copy(a_h.at[sidx.at[pl.ds(ro0 * 128, 128)]],
                                   buf_a0, sem0)
            g0b = pltpu.async_copy(b_h.at[didx.at[pl.ds(ro0 * 128, 128)]],
                                   buf_b0, sem0)
            g1a = pltpu.async_copy(a_h.at[sidx.at[pl.ds(ro1 * 128, 128)]],
                                   buf_a1, sem1)
            g1b = pltpu.async_copy(b_h.at[didx.at[pl.ds(ro1 * 128, 128)]],
                                   buf_b1, sem1)
            make_vec(ro0, vbuf0)
            wv0 = pltpu.async_copy(vbuf0, vec_h.at[pl.ds(r0 * 4, 512)], semv)
            make_vec(ro1, vbuf1)
            wv1 = pltpu.async_copy(vbuf1, vec_h.at[pl.ds(r1 * 4, 512)], semv)
            g0a.wait(); g0b.wait()
            add_into(buf_a0, buf_b0)
            w0 = pltpu.async_copy(buf_a0, esum_h.at[pl.ds(r0, 128)], sem0)
            g1a.wait(); g1b.wait()
            add_into(buf_a1, buf_b1)
            w1 = pltpu.async_copy(buf_a1, esum_h.at[pl.ds(r1, 128)], sem1)
            w0.wait(); w1.wait()
            wv0.wait(); wv1.wait()
        return 0

    lax.fori_loop(0, n_oct, octet, 0)


def _edge_kernel(es_ref, vec_ref, dstc_ref, wrbf_ref, bmsg_ref,
                 wf_ref, m_ref, u_ref):
    f32 = _F32
    i = pl.program_id(0)
    vec = vec_ref[:]
    d2 = jnp.sum(vec * vec, axis=1, keepdims=True)
    dist = jnp.sqrt(d2) + 1e-8
    centers = jax.lax.broadcasted_iota(jnp.int32, (E_BLK, N_RBF), 1).astype(f32) * (
        CUTOFF / (N_RBF - 1))
    rbf = jnp.exp(-((dist - centers) ** 2) / 0.5)
    rbf_w = jnp.dot(rbf, wrbf_ref[:], preferred_element_type=f32)
    mpre = es_ref[:] + rbf_w + bmsg_ref[:]
    m = mpre / (1.0 + jnp.exp(-mpre))
    m_ref[:] = m
    fcoef = jnp.sum(m * wf_ref[:], axis=1, keepdims=True)
    fv = fcoef * (vec / dist)

    dstc = dstc_ref[:]
    r_e = dstc % 128
    q_e = dstc // 128
    oh_r = (jax.lax.broadcasted_iota(jnp.int32, (E_BLK, 128), 1) == r_e
            ).astype(f32)
    oh_q = (jax.lax.broadcasted_iota(jnp.int32, (E_BLK, NQ), 1) == q_e
            ).astype(f32)
    dims = (((0,), (0,)), ((), ()))

    @pl.when(i == 0)
    def _():
        u_ref[:] = jnp.zeros_like(u_ref)

    for c in range(3):
        uc = jax.lax.dot_general(oh_r, oh_q * fv[:, c:c + 1], dims,
                                 preferred_element_type=f32)
        u_ref[c] += uc


def _sc_scatter(m_h, dst_h, z128_h, oagg_h,
                didx, mbuf0, mbuf1, agg_sp, sem0, sem1):
    cid = lax.axis_index("c")
    sid = lax.axis_index("s")
    wid = sid * 2 + cid
    base = wid * RPW
    row0 = sid * A_SLC
    pltpu.sync_copy(z128_h.at[pl.ds(row0, A_SLC)], agg_sp.at[pl.ds(row0, A_SLC)])
    pltpu.sync_copy(dst_h.at[pl.ds(base, RPW)], didx)
    plsc.subcore_barrier()

    def pair(jj, _):
        j0 = 2 * jj
        j1 = j0 + 1
        c0 = pltpu.async_copy(m_h.at[pl.ds((base + j0) * 128, 128)], mbuf0, sem0)
        c1 = pltpu.async_copy(m_h.at[pl.ds((base + j1) * 128, 128)], mbuf1, sem1)
        c0.wait()
        pltpu.sync_copy(mbuf0, agg_sp.at[didx.at[j0]], add=True)
        c1.wait()
        pltpu.sync_copy(mbuf1, agg_sp.at[didx.at[j1]], add=True)
        return 0

    lax.fori_loop(0, RPW // 2, pair, 0)
    plsc.subcore_barrier()
    pltpu.sync_copy(agg_sp.at[pl.ds(row0, A_SLC)],
                    oagg_h.at[cid, pl.ds(row0, A_SLC)])


def _out_kernel(h_ref, agg0_ref, agg1_ref,
                wu1_ref, wu2_ref, bu_ref, watom_ref, ba_ref,
                out_ref):
    f32 = _F32
    agg = agg0_ref[0] + agg1_ref[0]
    pre = (jnp.dot(h_ref[:], wu1_ref[:], preferred_element_type=f32)
           + jnp.dot(agg, wu2_ref[:], preferred_element_type=f32)
           + bu_ref[:])
    h2 = pre / (1.0 + jnp.exp(-pre))
    out_ref[:] = jnp.dot(h2, watom_ref[:], preferred_element_type=f32) + ba_ref[:]


def kernel(z, pred_frac_coords, pred_atom_types, num_atoms, lengths, angles,
           edge_index, atom_emb, W_z, b_z, W_msg, b_msg, W_upd, b_upd, W_f,
           W_atom, b_atom):
    f32 = _F32
    n_cryst = z.shape[0]
    n_atoms = pred_frac_coords.shape[0]
    n_edges = edge_index.shape[1]

    types2d = jnp.clip(pred_atom_types, 0, MAXA).astype(jnp.int32).reshape(-1, 1)
    types2d = jnp.pad(types2d, ((0, A_PAD - n_atoms), (0, 0)))
    frac4 = jnp.pad(pred_frac_coords.astype(f32), ((0, A_PAD - n_atoms), (0, 1)))
    z_p = jnp.pad(z.astype(f32), ((0, C_PAD - n_cryst), (0, 0)))
    len4 = jnp.pad(lengths.astype(f32), ((0, C_PAD - n_cryst), (0, 1)))
    ang4 = jnp.pad(angles.astype(f32), ((0, C_PAD - n_cryst), (0, 1)))
    emb_p = jnp.pad(atom_emb.astype(f32), ((0, 128 - (MAXA + 1)), (0, 0)))
    w_src = W_msg[:HID]
    w_dst = W_msg[HID:2 * HID]
    w_rbf = W_msg[2 * HID:]
    bz_r = b_z.reshape(1, HID)
    bmsg_r = b_msg.reshape(1, HID)
    wf_r = W_f.reshape(1, HID)
    wu1 = W_upd[:HID]
    wu2 = W_upd[HID:]
    bu_r = b_upd.reshape(1, HID)
    watom_p = jnp.pad(W_atom.astype(f32), ((0, 0), (0, 128 - MAXA)))
    ba_p = jnp.pad(b_atom.astype(f32), (0, 128 - MAXA)).reshape(1, 128)

    src_p = jnp.pad(edge_index[0].astype(jnp.int32), (0, E_PAD - n_edges),
                    constant_values=PAD_ATOM)
    dst_p = jnp.pad(edge_index[1].astype(jnp.int32), (0, E_PAD - n_edges),
                    constant_values=PAD_ATOM)
    src2d = src_p.reshape(E_ROWS, 128)
    dst2d = dst_p.reshape(E_ROWS, 128)
    dstc = dst_p.reshape(E_PAD, 1)
    zeros128 = jnp.zeros((A_PAD, HID), f32)

    n_ab = A_PAD // A_BLK

    def full(shp):
        return pl.BlockSpec(shp, lambda i: tuple(0 for _ in shp))

    h, a_tab, b_tab, cart4 = pl.pallas_call(
        _atom_kernel,
        grid=(n_ab,),
        in_specs=[
            pl.BlockSpec((A_BLK, 1), lambda i: (i, 0)),
            pl.BlockSpec((A_BLK, 4), lambda i: (i, 0)),
            pl.BlockSpec((C_BLK, LAT), lambda i: (i, 0)),
            pl.BlockSpec((C_BLK, 4), lambda i: (i, 0)),
            pl.BlockSpec((C_BLK, 4), lambda i: (i, 0)),
            full((128, HID)),
            full((LAT, HID)),
            full((1, HID)),
            full((HID, HID)),
            full((HID, HID)),
        ],
        out_specs=[
            pl.BlockSpec((A_BLK, HID), lambda i: (i, 0)),
            pl.BlockSpec((A_BLK, HID), lambda i: (i, 0)),
            pl.BlockSpec((A_BLK, HID), lambda i: (i, 0)),
            pl.BlockSpec((A_BLK, 4), lambda i: (i, 0)),
        ],
        out_shape=[
            jax.ShapeDtypeStruct((A_PAD, HID), f32),
            jax.ShapeDtypeStruct((A_PAD, HID), f32),
            jax.ShapeDtypeStruct((A_PAD, HID), f32),
            jax.ShapeDtypeStruct((A_PAD, 4), f32),
        ],
    )(types2d, frac4, z_p, len4, ang4, emb_p, W_z, bz_r, w_src, w_dst)

    cart_flat = cart4[:A_CART].reshape(A_CART * 4)

    mesh = plsc.VectorSubcoreMesh(core_axis_name="c", subcore_axis_name="s")
    gather_fn = functools.partial(
        pl.kernel,
        out_type=[
            jax.ShapeDtypeStruct((E_PAD, HID), f32),
            jax.ShapeDtypeStruct((E_PAD * 4,), f32),
        ],
        mesh=mesh,
        compiler_params=pltpu.CompilerParams(needs_layout_passes=False),
        scratch_types=[
            pltpu.VMEM((1024,), jnp.int32),
            pltpu.VMEM((1024,), jnp.int32),
            pltpu.VMEM((2, 16), jnp.int32),
            pltpu.VMEM((128, HID), f32),
            pltpu.VMEM((128, HID), f32),
            pltpu.VMEM((128, HID), f32),
            pltpu.VMEM((128, HID), f32),
            pltpu.VMEM((A_CART * 4,), f32),
            pltpu.VMEM((512,), f32),
            pltpu.VMEM((512,), f32),
            pltpu.SemaphoreType.DMA,
            pltpu.SemaphoreType.DMA,
            pltpu.SemaphoreType.DMA,
            pltpu.SemaphoreType.DMA,
        ],
    )(_sc_gather)
    lane_consts = jnp.array([[0, 0, 0, 0, 1, 1, 1, 1, 2, 2, 2, 2, 3, 3, 3, 3],
                             [0, 1, 2, 3, 0, 1, 2, 3, 0, 1, 2, 3, 0, 1, 2, 3]],
                            jnp.int32)
    e_sum, vec_flat = gather_fn(a_tab, b_tab, cart_flat, src_p, dst_p, lane_consts)
    vec4 = vec_flat.reshape(E_PAD, 4)

    n_eb = E_PAD // E_BLK
    m, u_acc = pl.pallas_call(
        _edge_kernel,
        grid=(n_eb,),
        in_specs=[
            pl.BlockSpec((E_BLK, HID), lambda i: (i, 0)),
            pl.BlockSpec((E_BLK, 4), lambda i: (i, 0)),
            pl.BlockSpec((E_BLK, 1), lambda i: (i, 0)),
            full((N_RBF, HID)),
            full((1, HID)),
            full((1, HID)),
        ],
        out_specs=[
            pl.BlockSpec((E_BLK, HID), lambda i: (i, 0)),
            pl.BlockSpec((3, 128, NQ), lambda i: (0, 0, 0)),
        ],
        out_shape=[
            jax.ShapeDtypeStruct((E_PAD, HID), f32),
            jax.ShapeDtypeStruct((3, 128, NQ), f32),
        ],
    )(e_sum, vec4, dstc, w_rbf, bmsg_r, wf_r)

    scatter_fn = functools.partial(
        pl.kernel,
        out_type=jax.ShapeDtypeStruct((2, A_PAD, HID), f32),
        mesh=mesh,
        scratch_types=[
            pltpu.VMEM((RPW, 128), jnp.int32),
            pltpu.VMEM((128, HID), f32),
            pltpu.VMEM((128, HID), f32),
            pltpu.VMEM_SHARED((A_PAD, HID), f32),
            pltpu.SemaphoreType.DMA,
            pltpu.SemaphoreType.DMA,
        ],
    )(_sc_scatter)
    agg_part = scatter_fn(m, dst2d, zeros128)

    out_p = pl.pallas_call(
        _out_kernel,
        grid=(n_ab,),
        in_specs=[
            pl.BlockSpec((A_BLK, HID), lambda i: (i, 0)),
            pl.BlockSpec((1, A_BLK, HID), lambda i: (0, i, 0)),
            pl.BlockSpec((1, A_BLK, HID), lambda i: (1, i, 0)),
            full((HID, HID)),
            full((HID, HID)),
            full((1, HID)),
            full((HID, 128)),
            full((1, 128)),
        ],
        out_specs=pl.BlockSpec((A_BLK, 128), lambda i: (i, 0)),
        out_shape=jax.ShapeDtypeStruct((A_PAD, 128), f32),
    )(h, agg_part, agg_part, wu1, wu2, bu_r, watom_p, ba_p)

    pred_cart = jnp.transpose(u_acc, (2, 1, 0)).reshape(A_PAD, 3)[:n_atoms]
    pred_atom_types_out = out_p[:n_atoms, :MAXA]
    return (pred_cart, pred_atom_types_out)


# speedup vs baseline: 1.1634x; 1.0005x over previous
import functools
import math

import jax
import jax.numpy as jnp
from jax import lax
from jax.experimental import pallas as pl
from jax.experimental.pallas import tpu as pltpu
from jax.experimental.pallas import tpu_sc as plsc

HID = 128
LAT = 256
N_RBF = 16
CUTOFF = 6.0
MAXA = 100

C_PAD = 512
APC = 20
A_PAD = C_PAD * APC
C_BLK = 128
A_BLK = C_BLK * APC

E_ROWS = 2560
E_PAD = E_ROWS * 128
E_BLK = 4096
PAD_ATOM = 10008
A_CART = 10048
NQ = A_PAD // 128

NW = 32
RPW = E_ROWS // NW
SPAIR = E_ROWS // 16
R_C0 = 120
R_C1 = SPAIR - R_C0
TPS = 16
A_SLC = A_PAD // TPS

_F32 = jnp.float32


def _atom_kernel(types_ref, frac_ref, z_ref, len_ref, ang_ref, emb_ref,
                 wz_ref, bz_ref, wsrc_ref, wdst_ref,
                 h_ref, a_ref, b_ref, cart_ref):
    f32 = _F32
    lanes_a = jax.lax.broadcasted_iota(jnp.int32, (A_BLK, 128), 1)
    t = types_ref[:]
    oh_t = (lanes_a == t).astype(f32)
    emb = jnp.dot(oh_t, emb_ref[:], preferred_element_type=f32)

    zw = jnp.dot(z_ref[:], wz_ref[:], preferred_element_type=f32)
    rows = jax.lax.broadcasted_iota(jnp.int32, (A_BLK, 128), 0)
    oh_c = (lanes_a == rows // APC).astype(f32)
    zw_at = jnp.dot(oh_c, zw, preferred_element_type=f32)

    h = emb + zw_at + bz_ref[:]
    h_ref[:] = h
    a_ref[:] = jnp.dot(h, wsrc_ref[:], preferred_element_type=f32)
    b_ref[:] = jnp.dot(h, wdst_ref[:], preferred_element_type=f32)

    ang = ang_ref[:] * (math.pi / 180.0)
    cos_a = jnp.cos(ang[:, 0:1])
    cos_b = jnp.cos(ang[:, 1:2])
    cos_g = jnp.cos(ang[:, 2:3])
    sin_g = jnp.clip(jnp.sin(ang[:, 2:3]), 1e-6, None)
    a_len = len_ref[:, 0:1]
    b_len = len_ref[:, 1:2]
    c_len = len_ref[:, 2:3]
    zeros = jnp.zeros_like(a_len)
    cx = c_len * cos_b
    cy = c_len * (cos_a - cos_b * cos_g) / sin_g
    cz = jnp.sqrt(jnp.clip(c_len * c_len - cx * cx - cy * cy, 1e-6, None))
    lrow = jnp.concatenate(
        [a_len, zeros, zeros,
         b_len * cos_g, b_len * sin_g, zeros,
         cx, cy, cz,
         zeros, zeros, zeros, zeros, zeros, zeros, zeros], axis=1)
    l_at = jnp.dot(oh_c, lrow, preferred_element_type=f32)

    f = frac_ref[:]
    f0, f1, f2 = f[:, 0:1], f[:, 1:2], f[:, 2:3]
    cart_x = f0 * l_at[:, 0:1] + f1 * l_at[:, 3:4] + f2 * l_at[:, 6:7]
    cart_y = f0 * l_at[:, 1:2] + f1 * l_at[:, 4:5] + f2 * l_at[:, 7:8]
    cart_z = f0 * l_at[:, 2:3] + f1 * l_at[:, 5:6] + f2 * l_at[:, 8:9]
    zpad = jnp.zeros((A_BLK, 1), f32)
    cart_ref[:] = jnp.concatenate([cart_x, cart_y, cart_z, zpad], axis=1)


def _sc_gather(a_h, b_h, cart_h, src_h, dst_h, lanec_h,
               esum_h, vec_h,
               sidx, didx, consts, buf_a0, buf_b0, buf_a1, buf_b1,
               cartv, vbuf0, vbuf1, sem0, sem1, semv, semi):
    f32 = _F32
    i32 = jnp.int32
    cid = lax.axis_index("c")
    sid = lax.axis_index("s")
    base = sid * SPAIR + cid * R_C0
    n_oct = (R_C0 // 8) + cid * ((R_C1 - R_C0) // 8)
    pltpu.sync_copy(cart_h, cartv)
    pltpu.sync_copy(lanec_h, consts)

    def make_vec(ro, vbuf):
        ld4 = consts[0]
        lm4 = consts[1]
        for k in range(32):
            ei = ld4 + (k * 4 + ro * 128)
            sa = plsc.load_gather(sidx, [ei])
            da = plsc.load_gather(didx, [ei])
            cs = plsc.load_gather(cartv, [sa * 4 + lm4])
            cd = plsc.load_gather(cartv, [da * 4 + lm4])
            vbuf[pl.ds(k * 16, 16)] = cs - cd

    def add_into(dst_buf, src_buf):
        def row(j2, _):
            for k in range(8):
                dst_buf[j2, pl.ds(k * 16, 16)] = (
                    dst_buf[j2, pl.ds(k * 16, 16)]
                    + src_buf[j2, pl.ds(k * 16, 16)])
            return 0
        lax.fori_loop(0, 128, row, 0)

    def octet(oo, _):
        g = base + 8 * oo
        ci0 = pltpu.async_copy(src_h.at[pl.ds(g * 128, 1024)], sidx, semi)
        ci1 = pltpu.async_copy(dst_h.at[pl.ds(g * 128, 1024)], didx, semi)
        ci0.wait(); ci1.wait()
        for pp in range(4):
            ro0 = 2 * pp
            ro1 = ro0 + 1
            r0 = (g + ro0) * 128
            r1 = (g + ro1) * 128
            g0a = pltpu.async_copy(a_h.at[sidx.at[pl.ds(ro0 * 128, 128)]],
                                   buf_a0, sem0)
            g0b = pltpu.async_copy(b_h.at[didx.at[pl.ds(ro0 * 128, 128)]],
                                   buf_b0, sem0)
            g1a = pltpu.async_copy(a_h.at[sidx.at[pl.ds(ro1 * 128, 128)]],
                                   buf_a1, sem1)
            g1b = pltpu.async_copy(b_h.at[didx.at[pl.ds(ro1 * 128, 128)]],
                                   buf_b1, sem1)
            make_vec(ro0, vbuf0)
            wv0 = pltpu.async_copy(vbuf0, vec_h.at[pl.ds(r0 * 4, 512)], semv)
            make_vec(ro1, vbuf1)
            wv1 = pltpu.async_copy(vbuf1, vec_h.at[pl.ds(r1 * 4, 512)], semv)
            g0a.wait(); g0b.wait()
            add_into(buf_a0, buf_b0)
            w0 = pltpu.async_copy(buf_a0, esum_h.at[pl.ds(r0, 128)], sem0)
            g1a.wait(); g1b.wait()
            add_into(buf_a1, buf_b1)
            w1 = pltpu.async_copy(buf_a1, esum_h.at[pl.ds(r1, 128)], sem1)
            w0.wait(); w1.wait()
            wv0.wait(); wv1.wait()
        return 0

    lax.fori_loop(0, n_oct, octet, 0)


def _edge_kernel(es_ref, vec_ref, dstc_ref, wrbf_ref, bmsg_ref,
                 wf_ref, m_ref, u_ref):
    f32 = _F32
    i = pl.program_id(0)
    vec = vec_ref[:]
    d2 = jnp.sum(vec * vec, axis=1, keepdims=True)
    dist = jnp.sqrt(d2) + 1e-8
    centers = jax.lax.broadcasted_iota(jnp.int32, (E_BLK, N_RBF), 1).astype(f32) * (
        CUTOFF / (N_RBF - 1))
    rbf = jnp.exp(-((dist - centers) ** 2) / 0.5)
    rbf_w = jnp.dot(rbf, wrbf_ref[:], preferred_element_type=f32)
    mpre = es_ref[:] + rbf_w + bmsg_ref[:]
    m = mpre / (1.0 + jnp.exp(-mpre))
    m_ref[:] = m
    fcoef = jnp.sum(m * wf_ref[:], axis=1, keepdims=True)
    fv = fcoef * (vec / dist)

    dstc = dstc_ref[:]
    r_e = dstc % 128
    q_e = dstc // 128
    oh_r = (jax.lax.broadcasted_iota(jnp.int32, (E_BLK, 128), 1) == r_e
            ).astype(f32)
    oh_q = (jax.lax.broadcasted_iota(jnp.int32, (E_BLK, NQ), 1) == q_e
            ).astype(f32)
    dims = (((0,), (0,)), ((), ()))

    @pl.when(i == 0)
    def _():
        u_ref[:] = jnp.zeros_like(u_ref)

    for c in range(3):
        uc = jax.lax.dot_general(oh_r, oh_q * fv[:, c:c + 1], dims,
                                 preferred_element_type=f32)
        u_ref[c] += uc


def _sc_scatter(m_h, dst_h, z128_h, oagg_h,
                didx, mbuf0, mbuf1, agg_sp, sem0, sem1):
    cid = lax.axis_index("c")
    sid = lax.axis_index("s")
    wid = sid * 2 + cid
    base = wid * RPW
    row0 = sid * A_SLC
    pltpu.sync_copy(z128_h.at[pl.ds(row0, A_SLC)], agg_sp.at[pl.ds(row0, A_SLC)])
    pltpu.sync_copy(dst_h.at[pl.ds(base, RPW)], didx)
    plsc.subcore_barrier()

    def pair(jj, _):
        j0 = 2 * jj
        j1 = j0 + 1
        c0 = pltpu.async_copy(m_h.at[pl.ds((base + j0) * 128, 128)], mbuf0, sem0)
        c1 = pltpu.async_copy(m_h.at[pl.ds((base + j1) * 128, 128)], mbuf1, sem1)
        c0.wait()
        pltpu.sync_copy(mbuf0, agg_sp.at[didx.at[j0]], add=True)
        c1.wait()
        pltpu.sync_copy(mbuf1, agg_sp.at[didx.at[j1]], add=True)
        return 0

    lax.fori_loop(0, RPW // 2, pair, 0)
    plsc.subcore_barrier()
    pltpu.sync_copy(agg_sp.at[pl.ds(row0, A_SLC)],
                    oagg_h.at[cid, pl.ds(row0, A_SLC)])


def _out_kernel(h_ref, agg0_ref, agg1_ref,
                wu1_ref, wu2_ref, bu_ref, watom_ref, ba_ref,
                out_ref):
    f32 = _F32
    agg = agg0_ref[0] + agg1_ref[0]
    pre = (jnp.dot(h_ref[:], wu1_ref[:], preferred_element_type=f32)
           + jnp.dot(agg, wu2_ref[:], preferred_element_type=f32)
           + bu_ref[:])
    h2 = pre / (1.0 + jnp.exp(-pre))
    out_ref[:] = jnp.dot(h2, watom_ref[:], preferred_element_type=f32) + ba_ref[:]


def kernel(z, pred_frac_coords, pred_atom_types, num_atoms, lengths, angles,
           edge_index, atom_emb, W_z, b_z, W_msg, b_msg, W_upd, b_upd, W_f,
           W_atom, b_atom):
    f32 = _F32
    n_cryst = z.shape[0]
    n_atoms = pred_frac_coords.shape[0]
    n_edges = edge_index.shape[1]

    types2d = jnp.clip(pred_atom_types, 0, MAXA).astype(jnp.int32).reshape(-1, 1)
    types2d = jnp.pad(types2d, ((0, A_PAD - n_atoms), (0, 0)))
    frac4 = jnp.pad(pred_frac_coords.astype(f32), ((0, A_PAD - n_atoms), (0, 1)))
    z_p = jnp.pad(z.astype(f32), ((0, C_PAD - n_cryst), (0, 0)))
    len4 = jnp.pad(lengths.astype(f32), ((0, C_PAD - n_cryst), (0, 1)))
    ang4 = jnp.pad(angles.astype(f32), ((0, C_PAD - n_cryst), (0, 1)))
    emb_p = jnp.pad(atom_emb.astype(f32), ((0, 128 - (MAXA + 1)), (0, 0)))
    w_src = W_msg[:HID]
    w_dst = W_msg[HID:2 * HID]
    w_rbf = W_msg[2 * HID:]
    bz_r = b_z.reshape(1, HID)
    bmsg_r = b_msg.reshape(1, HID)
    wf_r = W_f.reshape(1, HID)
    wu1 = W_upd[:HID]
    wu2 = W_upd[HID:]
    bu_r = b_upd.reshape(1, HID)
    watom_p = jnp.pad(W_atom.astype(f32), ((0, 0), (0, 128 - MAXA)))
    ba_p = jnp.pad(b_atom.astype(f32), (0, 128 - MAXA)).reshape(1, 128)

    src_p = jnp.pad(edge_index[0].astype(jnp.int32), (0, E_PAD - n_edges),
                    constant_values=PAD_ATOM)
    dst_p = jnp.pad(edge_index[1].astype(jnp.int32), (0, E_PAD - n_edges),
                    constant_values=PAD_ATOM)
    src2d = src_p.reshape(E_ROWS, 128)
    dst2d = dst_p.reshape(E_ROWS, 128)
    dstc = dst_p.reshape(E_PAD, 1)
    zeros128 = jnp.zeros((A_PAD, HID), f32)

    n_ab = A_PAD // A_BLK

    def full(shp):
        return pl.BlockSpec(shp, lambda i: tuple(0 for _ in shp))

    h, a_tab, b_tab, cart4 = pl.pallas_call(
        _atom_kernel,
        grid=(n_ab,),
        in_specs=[
            pl.BlockSpec((A_BLK, 1), lambda i: (i, 0)),
            pl.BlockSpec((A_BLK, 4), lambda i: (i, 0)),
            pl.BlockSpec((C_BLK, LAT), lambda i: (i, 0)),
            pl.BlockSpec((C_BLK, 4), lambda i: (i, 0)),
            pl.BlockSpec((C_BLK, 4), lambda i: (i, 0)),
            full((128, HID)),
            full((LAT, HID)),
            full((1, HID)),
            full((HID, HID)),
            full((HID, HID)),
        ],
        out_specs=[
            pl.BlockSpec((A_BLK, HID), lambda i: (i, 0)),
            pl.BlockSpec((A_BLK, HID), lambda i: (i, 0)),
            pl.BlockSpec((A_BLK, HID), lambda i: (i, 0)),
            pl.BlockSpec((A_BLK, 4), lambda i: (i, 0)),
        ],
        out_shape=[
            jax.ShapeDtypeStruct((A_PAD, HID), f32),
            jax.ShapeDtypeStruct((A_PAD, HID), f32),
            jax.ShapeDtypeStruct((A_PAD, HID), f32),
            jax.ShapeDtypeStruct((A_PAD, 4), f32),
        ],
    )(types2d, frac4, z_p, len4, ang4, emb_p, W_z, bz_r, w_src, w_dst)

    cart_flat = cart4[:A_CART].reshape(A_CART * 4)

    mesh = plsc.VectorSubcoreMesh(core_axis_name="c", subcore_axis_name="s")
    gather_fn = functools.partial(
        pl.kernel,
        out_type=[
            jax.ShapeDtypeStruct((E_PAD, HID), f32),
            jax.ShapeDtypeStruct((E_PAD * 4,), f32),
        ],
        mesh=mesh,
        compiler_params=pltpu.CompilerParams(needs_layout_passes=False),
        scratch_types=[
            pltpu.VMEM((1024,), jnp.int32),
            pltpu.VMEM((1024,), jnp.int32),
            pltpu.VMEM((2, 16), jnp.int32),
            pltpu.VMEM((128, HID), f32),
            pltpu.VMEM((128, HID), f32),
            pltpu.VMEM((128, HID), f32),
            pltpu.VMEM((128, HID), f32),
            pltpu.VMEM((A_CART * 4,), f32),
            pltpu.VMEM((512,), f32),
            pltpu.VMEM((512,), f32),
            pltpu.SemaphoreType.DMA,
            pltpu.SemaphoreType.DMA,
            pltpu.SemaphoreType.DMA,
            pltpu.SemaphoreType.DMA,
        ],
    )(_sc_gather)
    lane_consts = jnp.array([[0, 0, 0, 0, 1, 1, 1, 1, 2, 2, 2, 2, 3, 3, 3, 3],
                             [0, 1, 2, 3, 0, 1, 2, 3, 0, 1, 2, 3, 0, 1, 2, 3]],
                            jnp.int32)
    e_sum, vec_flat = gather_fn(a_tab, b_tab, cart_flat, src_p, dst_p, lane_consts)
    vec4 = vec_flat.reshape(E_PAD, 4)

    n_eb = E_PAD // E_BLK
    m, u_acc = pl.pallas_call(
        _edge_kernel,
        grid=(n_eb,),
        in_specs=[
            pl.BlockSpec((E_BLK, HID), lambda i: (i, 0)),
            pl.BlockSpec((E_BLK, 4), lambda i: (i, 0)),
            pl.BlockSpec((E_BLK, 1), lambda i: (i, 0)),
            full((N_RBF, HID)),
            full((1, HID)),
            full((1, HID)),
        ],
        out_specs=[
            pl.BlockSpec((E_BLK, HID), lambda i: (i, 0)),
            pl.BlockSpec((3, 128, NQ), lambda i: (0, 0, 0)),
        ],
        out_shape=[
            jax.ShapeDtypeStruct((E_PAD, HID), f32),
            jax.ShapeDtypeStruct((3, 128, NQ), f32),
        ],
        compiler_params=pltpu.CompilerParams(fuse_transposed_lhs_in_matmul=True),
    )(e_sum, vec4, dstc, w_rbf, bmsg_r, wf_r)

    scatter_fn = functools.partial(
        pl.kernel,
        out_type=jax.ShapeDtypeStruct((2, A_PAD, HID), f32),
        mesh=mesh,
        scratch_types=[
            pltpu.VMEM((RPW, 128), jnp.int32),
            pltpu.VMEM((128, HID), f32),
            pltpu.VMEM((128, HID), f32),
            pltpu.VMEM_SHARED((A_PAD, HID), f32),
            pltpu.SemaphoreType.DMA,
            pltpu.SemaphoreType.DMA,
        ],
    )(_sc_scatter)
    agg_part = scatter_fn(m, dst2d, zeros128)

    out_p = pl.pallas_call(
        _out_kernel,
        grid=(n_ab,),
        in_specs=[
            pl.BlockSpec((A_BLK, HID), lambda i: (i, 0)),
            pl.BlockSpec((1, A_BLK, HID), lambda i: (0, i, 0)),
            pl.BlockSpec((1, A_BLK, HID), lambda i: (1, i, 0)),
            full((HID, HID)),
            full((HID, HID)),
            full((1, HID)),
            full((HID, 128)),
            full((1, 128)),
        ],
        out_specs=pl.BlockSpec((A_BLK, 128), lambda i: (i, 0)),
        out_shape=jax.ShapeDtypeStruct((A_PAD, 128), f32),
    )(h, agg_part, agg_part, wu1, wu2, bu_r, watom_p, ba_p)

    pred_cart = jnp.transpose(u_acc, (2, 1, 0)).reshape(A_PAD, 3)[:n_atoms]
    pred_atom_types_out = out_p[:n_atoms, :MAXA]
    return (pred_cart, pred_atom_types_out)
